# baseline (speedup 1.0000x reference)
import jax
import jax.numpy as jnp
from jax import lax
from jax.experimental import pallas as pl
from jax.experimental.pallas import tpu as pltpu
from jax.experimental.pallas import tpu_sc as plsc

_CAP = 50
_MEM = 128
_B = 4096
_ROWS = _B * _CAP

_NC = 2
_NS = 16
_NW = _NC * _NS
_BPW = _B // _NW
_RPW = _BPW * _CAP
_L = 16


def _sc_body(inputs_hbm, cnt_hbm, mem_hbm, out_hbm, cnt_out_hbm,
             cnt_v, idx_v, cnt1_v, rows_v, copy_sem, scat_sem):
    wid = lax.axis_index("s") * _NC + lax.axis_index("c")
    base = wid * _BPW
    srow = base * _CAP
    bulk = pltpu.async_copy(mem_hbm.at[pl.ds(srow, _RPW)],
                            out_hbm.at[pl.ds(srow, _RPW)], copy_sem)
    pltpu.sync_copy(cnt_hbm.at[pl.ds(base, _BPW)], cnt_v)
    pltpu.sync_copy(inputs_hbm.at[pl.ds(base, _BPW)], rows_v)
    for i in range(_BPW // _L):
        cv = cnt_v[pl.ds(i * _L, _L)]
        slot = lax.rem(cv, _CAP)
        brow = (base + i * _L) + lax.iota(jnp.int32, _L)
        idx_v[pl.ds(i * _L, _L)] = brow * _CAP + slot
        cnt1_v[pl.ds(i * _L, _L)] = cv + 1
    pltpu.sync_copy(cnt1_v, cnt_out_hbm.at[pl.ds(base, _BPW)])
    bulk.wait()
    pltpu.async_copy(rows_v, out_hbm.at[idx_v], scat_sem).wait()


_sc_write = pl.kernel(
    _sc_body,
    out_type=(jax.ShapeDtypeStruct((_ROWS, _MEM), jnp.float32),
              jax.ShapeDtypeStruct((_B,), jnp.int32)),
    mesh=plsc.VectorSubcoreMesh(core_axis_name="c", subcore_axis_name="s",
                                num_cores=_NC, num_subcores=_NS),
    scratch_types=[
        pltpu.VMEM((_BPW,), jnp.int32),
        pltpu.VMEM((_BPW,), jnp.int32),
        pltpu.VMEM((_BPW,), jnp.int32),
        pltpu.VMEM((_BPW, _MEM), jnp.float32),
        pltpu.SemaphoreType.DMA,
        pltpu.SemaphoreType.DMA,
    ],
)


def kernel(inputs, cnt, mem):
    cnt = cnt.astype(jnp.int32)
    memories2d, counter = _sc_write(inputs, cnt, mem.reshape(_ROWS, _MEM))
    memories = memories2d.reshape(_B, _CAP, _MEM)
    return (memories, counter, memories)


# speedup vs baseline: 6.6927x; 6.6927x over previous
import jax
import jax.numpy as jnp
from jax import lax
from jax.experimental import pallas as pl
from jax.experimental.pallas import tpu as pltpu
from jax.experimental.pallas import tpu_sc as plsc

_CAP = 50
_MEM = 128
_B = 4096
_ELEMS = _B * _CAP * _MEM

_NC = 2
_NS = 16
_NW = _NC * _NS
_BPW = _B // _NW
_EPW = _ELEMS // _NW
_L = 16

_RPC = 2
_CH = _RPC * _CAP * _MEM
_NCH = _EPW // _CH
_NB = 4


def _sc_body(inputs_hbm, cnt_hbm, mem_hbm, out_hbm, cnt_out_hbm,
             cnt_v, cnt1_v, lrow_v, rows_v, buf0, buf1, buf2, buf3,
             rd_sem0, rd_sem1, rd_sem2, rd_sem3,
             wr_sem0, wr_sem1, wr_sem2, wr_sem3):
    wid = lax.axis_index("s") * _NC + lax.axis_index("c")
    base = wid * _BPW
    sel = wid * _EPW
    bufs = (buf0, buf1, buf2, buf3)
    rd_sems = (rd_sem0, rd_sem1, rd_sem2, rd_sem3)
    wr_sems = (wr_sem0, wr_sem1, wr_sem2, wr_sem3)

    def rd(k):
        return pltpu.async_copy(
            mem_hbm.at[pl.ds(sel + k * _CH, _CH)], bufs[k % _NB],
            rd_sems[k % _NB])

    def wr(k):
        return pltpu.async_copy(
            bufs[k % _NB], out_hbm.at[pl.ds(sel + k * _CH, _CH)],
            wr_sems[k % _NB])

    rds = {0: rd(0), 1: rd(1)}

    pltpu.sync_copy(cnt_hbm.at[pl.ds(base, _BPW)], cnt_v)
    pltpu.sync_copy(inputs_hbm.at[pl.ds(base * _MEM, _BPW * _MEM)], rows_v)
    half = lax.rem(lax.iota(jnp.int32, _L), _RPC) * _CAP
    for i in range(_BPW // _L):
        cv = cnt_v[pl.ds(i * _L, _L)]
        cnt1_v[pl.ds(i * _L, _L)] = cv + 1
        lrow_v[pl.ds(i * _L, _L)] = half + lax.rem(cv, _CAP)
    pltpu.sync_copy(cnt1_v, cnt_out_hbm.at[pl.ds(base, _BPW)])

    wrs = {}
    for k in range(_NCH):
        if k + 2 < _NCH:
            if k >= 2:
                wrs[k - 2].wait()
            rds[k + 2] = rd(k + 2)
        rds[k].wait()
        for r in range(_RPC):
            g = _RPC * k + r
            off = lrow_v[pl.ds((g // _L) * _L, _L)][g % _L] * _MEM
            for j in range(_MEM // _L):
                bufs[k % _NB][pl.ds(off + j * _L, _L)] = (
                    rows_v[pl.ds(g * _MEM + j * _L, _L)])
        wrs[k] = wr(k)
    for k in range(_NCH - 2, _NCH):
        wrs[k].wait()


_sc_write = pl.kernel(
    _sc_body,
    out_type=(jax.ShapeDtypeStruct((_ELEMS,), jnp.float32),
              jax.ShapeDtypeStruct((_B,), jnp.int32)),
    mesh=plsc.VectorSubcoreMesh(core_axis_name="c", subcore_axis_name="s",
                                num_cores=_NC, num_subcores=_NS),
    scratch_types=[
        pltpu.VMEM((_BPW,), jnp.int32),
        pltpu.VMEM((_BPW,), jnp.int32),
        pltpu.VMEM((_BPW,), jnp.int32),
        pltpu.VMEM((_BPW * _MEM,), jnp.float32),
        pltpu.VMEM((_CH,), jnp.float32),
        pltpu.VMEM((_CH,), jnp.float32),
        pltpu.VMEM((_CH,), jnp.float32),
        pltpu.VMEM((_CH,), jnp.float32),
        pltpu.SemaphoreType.DMA,
        pltpu.SemaphoreType.DMA,
        pltpu.SemaphoreType.DMA,
        pltpu.SemaphoreType.DMA,
        pltpu.SemaphoreType.DMA,
        pltpu.SemaphoreType.DMA,
        pltpu.SemaphoreType.DMA,
        pltpu.SemaphoreType.DMA,
    ],
)


def kernel(inputs, cnt, mem):
    cnt = cnt.astype(jnp.int32)
    memories_flat, counter = _sc_write(inputs.reshape(-1), cnt,
                                       mem.reshape(-1))
    memories = memories_flat.reshape(_B, _CAP, _MEM)
    return (memories, counter, memories)


# speedup vs baseline: 6.7049x; 1.0018x over previous
import jax
import jax.numpy as jnp
from jax import lax
from jax.experimental import pallas as pl
from jax.experimental.pallas import tpu as pltpu
from jax.experimental.pallas import tpu_sc as plsc

_CAP = 50
_MEM = 128
_B = 4096
_ELEMS = _B * _CAP * _MEM

_NC = 2
_NS = 16
_NW = _NC * _NS
_BPW = _B // _NW
_EPW = _ELEMS // _NW
_L = 16

_RPC = 2
_CH = _RPC * _CAP * _MEM
_NCH = _EPW // _CH
_NB = 8
_RA = 4


def _sc_body(inputs_hbm, cnt_hbm, mem_hbm, out_hbm, cnt_out_hbm,
             cnt_v, cnt1_v, lrow_v, rows_v,
             buf0, buf1, buf2, buf3, buf4, buf5, buf6, buf7,
             rd_sem0, rd_sem1, rd_sem2, rd_sem3,
             rd_sem4, rd_sem5, rd_sem6, rd_sem7,
             wr_sem0, wr_sem1, wr_sem2, wr_sem3,
             wr_sem4, wr_sem5, wr_sem6, wr_sem7):
    wid = lax.axis_index("s") * _NC + lax.axis_index("c")
    base = wid * _BPW
    sel = wid * _EPW
    bufs = (buf0, buf1, buf2, buf3, buf4, buf5, buf6, buf7)
    rd_sems = (rd_sem0, rd_sem1, rd_sem2, rd_sem3,
               rd_sem4, rd_sem5, rd_sem6, rd_sem7)
    wr_sems = (wr_sem0, wr_sem1, wr_sem2, wr_sem3,
               wr_sem4, wr_sem5, wr_sem6, wr_sem7)

    def rd(k):
        return pltpu.async_copy(
            mem_hbm.at[pl.ds(sel + k * _CH, _CH)], bufs[k % _NB],
            rd_sems[k % _NB])

    def wr(k):
        return pltpu.async_copy(
            bufs[k % _NB], out_hbm.at[pl.ds(sel + k * _CH, _CH)],
            wr_sems[k % _NB])

    rds = {k: rd(k) for k in range(_RA)}

    pltpu.sync_copy(cnt_hbm.at[pl.ds(base, _BPW)], cnt_v)
    pltpu.sync_copy(inputs_hbm.at[pl.ds(base * _MEM, _BPW * _MEM)], rows_v)
    half = lax.rem(lax.iota(jnp.int32, _L), _RPC) * _CAP
    for i in range(_BPW // _L):
        cv = cnt_v[pl.ds(i * _L, _L)]
        cnt1_v[pl.ds(i * _L, _L)] = cv + 1
        lrow_v[pl.ds(i * _L, _L)] = half + lax.rem(cv, _CAP)
    pltpu.sync_copy(cnt1_v, cnt_out_hbm.at[pl.ds(base, _BPW)])

    wrs = {}
    for k in range(_NCH):
        if k + _RA < _NCH:
            if k + _RA >= _NB:
                wrs[k + _RA - _NB].wait()
            rds[k + _RA] = rd(k + _RA)
        rds[k].wait()
        for r in range(_RPC):
            g = _RPC * k + r
            off = lrow_v[pl.ds((g // _L) * _L, _L)][g % _L] * _MEM
            for j in range(_MEM // _L):
                bufs[k % _NB][pl.ds(off + j * _L, _L)] = (
                    rows_v[pl.ds(g * _MEM + j * _L, _L)])
        wrs[k] = wr(k)
    for k in range(_NCH - _NB, _NCH):
        wrs[k].wait()


_sc_write = pl.kernel(
    _sc_body,
    out_type=(jax.ShapeDtypeStruct((_ELEMS,), jnp.float32),
              jax.ShapeDtypeStruct((_B,), jnp.int32)),
    mesh=plsc.VectorSubcoreMesh(core_axis_name="c", subcore_axis_name="s",
                                num_cores=_NC, num_subcores=_NS),
    scratch_types=[
        pltpu.VMEM((_BPW,), jnp.int32),
        pltpu.VMEM((_BPW,), jnp.int32),
        pltpu.VMEM((_BPW,), jnp.int32),
        pltpu.VMEM((_BPW * _MEM,), jnp.float32),
    ] + [pltpu.VMEM((_CH,), jnp.float32)] * _NB
      + [pltpu.SemaphoreType.DMA] * (2 * _NB),
)


def kernel(inputs, cnt, mem):
    cnt = cnt.astype(jnp.int32)
    memories_flat, counter = _sc_write(inputs.reshape(-1), cnt,
                                       mem.reshape(-1))
    memories = memories_flat.reshape(_B, _CAP, _MEM)
    return (memories, counter, memories)


# speedup vs baseline: 12.1480x; 1.8118x over previous
import jax
import jax.numpy as jnp
from jax import lax
from jax.experimental import pallas as pl
from jax.experimental.pallas import tpu as pltpu
from jax.experimental.pallas import tpu_sc as plsc

_CAP = 50
_MEM = 128
_B = 4096

_NC = 2
_NS = 16
_NW = _NC * _NS
_BPW = _B // _NW
_L = 16

_BPC = 4
_NCH = _BPW // _BPC
_NB = 3


def _sc_body(inputs_hbm, cnt_hbm, mem_hbm, out_hbm, cnt_out_hbm,
             cnt_v, cnt1_v, slot_v, rows_v, buf0, buf1, buf2,
             rd_sem0, rd_sem1, rd_sem2, wr_sem0, wr_sem1, wr_sem2):
    wid = lax.axis_index("s") * _NC + lax.axis_index("c")
    base = wid * _BPW
    bufs = (buf0, buf1, buf2)
    rd_sems = (rd_sem0, rd_sem1, rd_sem2)
    wr_sems = (wr_sem0, wr_sem1, wr_sem2)

    def rd(k):
        return pltpu.async_copy(
            mem_hbm.at[pl.ds(base + k * _BPC, _BPC)], bufs[k % _NB],
            rd_sems[k % _NB])

    def wr(k):
        return pltpu.async_copy(
            bufs[k % _NB], out_hbm.at[pl.ds(base + k * _BPC, _BPC)],
            wr_sems[k % _NB])

    rds = {k: rd(k) for k in range(_NB)}

    pltpu.sync_copy(cnt_hbm.at[pl.ds(base, _BPW)], cnt_v)
    pltpu.sync_copy(inputs_hbm.at[pl.ds(base, _BPW)], rows_v)
    for i in range(_BPW // _L):
        cv = cnt_v[pl.ds(i * _L, _L)]
        cnt1_v[pl.ds(i * _L, _L)] = cv + 1
        slot_v[pl.ds(i * _L, _L)] = lax.rem(cv, _CAP)
    pltpu.sync_copy(cnt1_v, cnt_out_hbm.at[pl.ds(base, _BPW)])

    wrs = {}
    for k in range(_NCH):
        if k >= 1 and k + _NB - 1 < _NCH:
            wrs[k - 1].wait()
            rds[k + _NB - 1] = rd(k + _NB - 1)
        rds[k].wait()
        for r in range(_BPC):
            g = _BPC * k + r
            slot = slot_v[pl.ds((g // _L) * _L, _L)][g % _L]
            for j in range(_MEM // _L):
                bufs[k % _NB][r, slot, pl.ds(j * _L, _L)] = (
                    rows_v[g, pl.ds(j * _L, _L)])
        wrs[k] = wr(k)
    for k in range(_NCH - _NB, _NCH):
        wrs[k].wait()


_sc_write = pl.kernel(
    _sc_body,
    out_type=(jax.ShapeDtypeStruct((_B, _CAP, _MEM), jnp.float32),
              jax.ShapeDtypeStruct((_B,), jnp.int32)),
    mesh=plsc.VectorSubcoreMesh(core_axis_name="c", subcore_axis_name="s",
                                num_cores=_NC, num_subcores=_NS),
    scratch_types=[
        pltpu.VMEM((_BPW,), jnp.int32),
        pltpu.VMEM((_BPW,), jnp.int32),
        pltpu.VMEM((_BPW,), jnp.int32),
        pltpu.VMEM((_BPW, _MEM), jnp.float32),
    ] + [pltpu.VMEM((_BPC, _CAP, _MEM), jnp.float32)] * _NB
      + [pltpu.SemaphoreType.DMA] * (2 * _NB),
)


def kernel(inputs, cnt, mem):
    cnt = cnt.astype(jnp.int32)
    memories, counter = _sc_write(inputs, cnt, mem)
    return (memories, counter, memories)


# speedup vs baseline: 21.4796x; 1.7682x over previous
import jax
import jax.numpy as jnp
from jax import lax
from jax.experimental import pallas as pl
from jax.experimental.pallas import tpu as pltpu
from jax.experimental.pallas import tpu_sc as plsc

_CAP = 50
_MEM = 128
_B = 4096

_NC = 2
_NS = 16
_NW = _NC * _NS
_BPW = _B // _NW
_L = 16
_NG = _BPW // _L
_HB = 80

_NB = 4
_RA = 2


def _sc_body(inputs_hbm, cnt_hbm, mem_hbm, out_hbm, cnt_out_hbm,
             cnt_v, slot_v, rows_v, hist_v, offs_v, cur_v, list_v,
             buf0, buf1, buf2, buf3,
             rd_sem0, rd_sem1, rd_sem2, rd_sem3,
             wr_sem0, wr_sem1, wr_sem2, wr_sem3):
    wid = lax.axis_index("s") * _NC + lax.axis_index("c")
    base = wid * _BPW
    bufs = (buf0, buf1, buf2, buf3)
    rd_sems = (rd_sem0, rd_sem1, rd_sem2, rd_sem3)
    wr_sems = (wr_sem0, wr_sem1, wr_sem2, wr_sem3)

    def rd(s):
        return pltpu.async_copy(
            mem_hbm.at[s, pl.ds(base, _BPW)], bufs[s % _NB],
            rd_sems[s % _NB])

    def wr(s):
        return pltpu.async_copy(
            bufs[s % _NB], out_hbm.at[s, pl.ds(base, _BPW)],
            wr_sems[s % _NB])

    rds = {s: rd(s) for s in range(_RA)}

    pltpu.sync_copy(cnt_hbm.at[pl.ds(base, _BPW)], cnt_v)
    pltpu.sync_copy(inputs_hbm.at[pl.ds(base, _BPW)], rows_v)
    lanes = lax.iota(jnp.int32, _L)
    ones = jnp.ones((_L,), jnp.int32)
    zeros = jnp.zeros((_L,), jnp.int32)
    for i in range(_HB // _L):
        hist_v[pl.ds(i * _L, _L)] = zeros
    for i in range(_NG):
        cv = cnt_v[pl.ds(i * _L, _L)]
        sv = lax.rem(cv, _CAP)
        slot_v[pl.ds(i * _L, _L)] = sv
        cnt_v[pl.ds(i * _L, _L)] = cv + 1
        plsc.addupdate_scatter(hist_v, [sv], ones)
    pltpu.sync_copy(cnt_v, cnt_out_hbm.at[pl.ds(base, _BPW)])

    carry = jnp.int32(0)
    for i in range(4):
        h = hist_v[pl.ds(i * _L, _L)]
        c = plsc.cumsum(h)
        excl = c - h + carry
        offs_v[pl.ds(i * _L, _L)] = excl
        cur_v[pl.ds(i * _L, _L)] = excl
        carry = carry + c[_L - 1]

    lane0 = lanes == 0
    for gi in range(_NG):
        svg = slot_v[pl.ds(gi * _L, _L)]
        for l in range(_L):
            g = gi * _L + l
            s_g = svg[l]
            p = cur_v[pl.ds(s_g, _L)][0]
            plsc.store_scatter(list_v, [jnp.full((_L,), p, jnp.int32)],
                               jnp.full((_L,), g, jnp.int32), mask=lane0)
            plsc.store_scatter(cur_v, [jnp.full((_L,), s_g, jnp.int32)],
                               jnp.full((_L,), p + 1, jnp.int32),
                               mask=lane0)

    wrs = {}
    for s in range(_CAP):
        if s + _RA < _CAP:
            if s + _RA >= _NB:
                wrs[s + _RA - _NB].wait()
            rds[s + _RA] = rd(s + _RA)
        rds[s].wait()
        grp = (s // _L) * _L
        st = offs_v[pl.ds(grp, _L)][s % _L]
        n = hist_v[pl.ds(grp, _L)][s % _L]
        buf = bufs[s % _NB]

        def ov_body(t, c, st=st, buf=buf):
            g = list_v[pl.ds(st + t, _L)][0]
            for j in range(_MEM // _L):
                buf[g, pl.ds(j * _L, _L)] = rows_v[g, pl.ds(j * _L, _L)]
            return c

        lax.fori_loop(0, n, ov_body, jnp.int32(0))
        wrs[s] = wr(s)
    for s in range(_CAP - _NB, _CAP):
        wrs[s].wait()


_sc_write = pl.kernel(
    _sc_body,
    out_type=(jax.ShapeDtypeStruct((_CAP, _B, _MEM), jnp.float32),
              jax.ShapeDtypeStruct((_B,), jnp.int32)),
    mesh=plsc.VectorSubcoreMesh(core_axis_name="c", subcore_axis_name="s",
                                num_cores=_NC, num_subcores=_NS),
    compiler_params=pltpu.CompilerParams(needs_layout_passes=False),
    scratch_types=[
        pltpu.VMEM((_BPW,), jnp.int32),
        pltpu.VMEM((_BPW,), jnp.int32),
        pltpu.VMEM((_BPW, _MEM), jnp.float32),
        pltpu.VMEM((_HB,), jnp.int32),
        pltpu.VMEM((_HB,), jnp.int32),
        pltpu.VMEM((_HB,), jnp.int32),
        pltpu.VMEM((_BPW + _L,), jnp.int32),
    ] + [pltpu.VMEM((_BPW, _MEM), jnp.float32)] * _NB
      + [pltpu.SemaphoreType.DMA] * (2 * _NB),
)


def kernel(inputs, cnt, mem):
    cnt = cnt.astype(jnp.int32)
    mem_t = mem.transpose(1, 0, 2)
    out_t, counter = _sc_write(inputs, cnt, mem_t)
    memories = out_t.transpose(1, 0, 2)
    return (memories, counter, memories)


# speedup vs baseline: 23.4037x; 1.0896x over previous
import jax
import jax.numpy as jnp
from jax import lax
from jax.experimental import pallas as pl
from jax.experimental.pallas import tpu as pltpu
from jax.experimental.pallas import tpu_sc as plsc

_CAP = 50
_MEM = 128
_B = 4096

_NC = 2
_NS = 16
_NW = _NC * _NS
_BPW = _B // _NW
_L = 16
_NG = _BPW // _L
_HB = 80

_NB = 4
_RA = 2


def _sc_body(inputs_hbm, cnt_hbm, mem_hbm, out_hbm, cnt_out_hbm,
             cnt_v, slot_v, rows_v, hist_v, offs_v, cur_v, list_v,
             buf0, buf1, buf2, buf3,
             rd_sem0, rd_sem1, rd_sem2, rd_sem3,
             wr_sem0, wr_sem1, wr_sem2, wr_sem3):
    wid = lax.axis_index("s") * _NC + lax.axis_index("c")
    base = wid * _BPW
    bufs = (buf0, buf1, buf2, buf3)
    rd_sems = (rd_sem0, rd_sem1, rd_sem2, rd_sem3)
    wr_sems = (wr_sem0, wr_sem1, wr_sem2, wr_sem3)

    def rd(s):
        return pltpu.async_copy(
            mem_hbm.at[s, pl.ds(base, _BPW)], bufs[s % _NB],
            rd_sems[s % _NB])

    def wr(s):
        return pltpu.async_copy(
            bufs[s % _NB], out_hbm.at[s, pl.ds(base, _BPW)],
            wr_sems[s % _NB])

    rds = {s: rd(s) for s in range(_RA)}

    pltpu.sync_copy(cnt_hbm.at[pl.ds(base, _BPW)], cnt_v)
    pltpu.sync_copy(inputs_hbm.at[pl.ds(base, _BPW)], rows_v)
    lanes = lax.iota(jnp.int32, _L)
    ones = jnp.ones((_L,), jnp.int32)
    zeros = jnp.zeros((_L,), jnp.int32)
    for i in range(_HB // _L):
        hist_v[pl.ds(i * _L, _L)] = zeros
    for i in range(_NG):
        cv = cnt_v[pl.ds(i * _L, _L)]
        sv = lax.rem(cv, _CAP)
        slot_v[pl.ds(i * _L, _L)] = sv
        cnt_v[pl.ds(i * _L, _L)] = cv + 1
        plsc.addupdate_scatter(hist_v, [sv], ones)
    pltpu.sync_copy(cnt_v, cnt_out_hbm.at[pl.ds(base, _BPW)])

    carry = jnp.int32(0)
    for i in range(4):
        h = hist_v[pl.ds(i * _L, _L)]
        c = plsc.cumsum(h)
        excl = c - h + carry
        offs_v[pl.ds(i * _L, _L)] = excl
        cur_v[pl.ds(i * _L, _L)] = excl
        carry = carry + c[_L - 1]

    lane0 = lanes == 0
    for gi in range(_NG):
        svg = slot_v[pl.ds(gi * _L, _L)]
        for l in range(_L):
            g = gi * _L + l
            s_g = svg[l]
            p = cur_v[pl.ds(s_g, _L)][0]
            plsc.store_scatter(list_v, [jnp.full((_L,), p, jnp.int32)],
                               jnp.full((_L,), g, jnp.int32), mask=lane0)
            plsc.store_scatter(cur_v, [jnp.full((_L,), s_g, jnp.int32)],
                               jnp.full((_L,), p + 1, jnp.int32),
                               mask=lane0)

    wrs = {}
    for s in range(_CAP):
        if s + _RA < _CAP:
            if s + _RA >= _NB:
                wrs[s + _RA - _NB].wait()
            rds[s + _RA] = rd(s + _RA)
        rds[s].wait()
        grp = (s // _L) * _L
        st = offs_v[pl.ds(grp, _L)][s % _L]
        n = hist_v[pl.ds(grp, _L)][s % _L]
        buf = bufs[s % _NB]

        def ov_body(t, c, st=st, buf=buf):
            g = list_v[pl.ds(st + t, _L)][0]
            for j in range(_MEM // _L):
                buf[g, pl.ds(j * _L, _L)] = rows_v[g, pl.ds(j * _L, _L)]
            return c

        lax.fori_loop(0, n, ov_body, jnp.int32(0))
        wrs[s] = wr(s)
    for s in range(_CAP - _NB, _CAP):
        wrs[s].wait()


_sc_write = pl.kernel(
    _sc_body,
    out_type=(jax.ShapeDtypeStruct((_CAP, _B, _MEM), jnp.float32),
              jax.ShapeDtypeStruct((_B,), jnp.int32)),
    mesh=plsc.VectorSubcoreMesh(core_axis_name="c", subcore_axis_name="s",
                                num_cores=_NC, num_subcores=_NS),
    compiler_params=pltpu.CompilerParams(needs_layout_passes=False),
    scratch_types=[
        pltpu.VMEM((_BPW,), jnp.int32),
        pltpu.VMEM((_BPW,), jnp.int32),
        pltpu.VMEM((_BPW, _MEM), jnp.float32),
        pltpu.VMEM((_HB,), jnp.int32),
        pltpu.VMEM((_HB,), jnp.int32),
        pltpu.VMEM((_HB,), jnp.int32),
        pltpu.VMEM((_BPW + _L,), jnp.int32),
    ] + [pltpu.VMEM((_BPW, _MEM), jnp.float32)] * _NB
      + [pltpu.SemaphoreType.DMA] * (2 * _NB),
)


_TBB = 256


def _tc_dup_body(cnt_ref, x_ref, mem_ref, o_ref):
    slot = lax.rem(cnt_ref[...], _CAP)
    sel = lax.broadcasted_iota(jnp.int32, (_CAP, _TBB, _MEM), 0)
    o_ref[...] = jnp.where(sel == slot[None, :, None],
                           x_ref[...][None, :, :], mem_ref[...])


_tc_dup = pl.pallas_call(
    _tc_dup_body,
    grid=(_B // _TBB,),
    in_specs=[
        pl.BlockSpec((_TBB,), lambda i: (i,)),
        pl.BlockSpec((_TBB, _MEM), lambda i: (i, 0)),
        pl.BlockSpec((_CAP, _TBB, _MEM), lambda i: (0, i, 0)),
    ],
    out_specs=pl.BlockSpec((_CAP, _TBB, _MEM), lambda i: (0, i, 0)),
    out_shape=jax.ShapeDtypeStruct((_CAP, _B, _MEM), jnp.float32),
)


def kernel(inputs, cnt, mem):
    cnt = cnt.astype(jnp.int32)
    mem_t = mem.transpose(1, 0, 2)
    out_t, counter = _sc_write(inputs, cnt, mem_t)
    memories = out_t.transpose(1, 0, 2)
    dup = _tc_dup(cnt, inputs, mem_t).transpose(1, 0, 2)
    return (memories, counter, dup)


# speedup vs baseline: 24.5982x; 1.0510x over previous
import jax
import jax.numpy as jnp
from jax import lax
from jax.experimental import pallas as pl
from jax.experimental.pallas import tpu as pltpu
from jax.experimental.pallas import tpu_sc as plsc

_CAP = 50
_MEM = 128
_B = 4096

_NC = 2
_NS = 16
_NW = _NC * _NS
_BPW = _B // _NW
_L = 16
_NG = _BPW // _L
_HB = 80

_NB = 4
_RA = 2


def _sc_body(inputs_hbm, cnt_hbm, mem_hbm, out_hbm, out2_hbm, cnt_out_hbm,
             cnt_v, slot_v, rows_v, hist_v, offs_v, cur_v, list_v,
             buf0, buf1, buf2, buf3,
             rd_sem0, rd_sem1, rd_sem2, rd_sem3,
             wr_sem0, wr_sem1, wr_sem2, wr_sem3,
             w2_sem0, w2_sem1, w2_sem2, w2_sem3):
    wid = lax.axis_index("s") * _NC + lax.axis_index("c")
    base = wid * _BPW
    bufs = (buf0, buf1, buf2, buf3)
    rd_sems = (rd_sem0, rd_sem1, rd_sem2, rd_sem3)
    wr_sems = (wr_sem0, wr_sem1, wr_sem2, wr_sem3)
    w2_sems = (w2_sem0, w2_sem1, w2_sem2, w2_sem3)

    def rd(s):
        return pltpu.async_copy(
            mem_hbm.at[s, pl.ds(base, _BPW)], bufs[s % _NB],
            rd_sems[s % _NB])

    class _Wr2:
        def __init__(self, s):
            self.a = pltpu.async_copy(
                bufs[s % _NB], out_hbm.at[s, pl.ds(base, _BPW)],
                wr_sems[s % _NB])
            self.b = pltpu.async_copy(
                bufs[s % _NB], out2_hbm.at[s, pl.ds(base, _BPW)],
                w2_sems[s % _NB])

        def wait(self):
            self.a.wait()
            self.b.wait()

    wr = _Wr2

    rds = {s: rd(s) for s in range(_RA)}

    pltpu.sync_copy(cnt_hbm.at[pl.ds(base, _BPW)], cnt_v)
    pltpu.sync_copy(inputs_hbm.at[pl.ds(base, _BPW)], rows_v)
    lanes = lax.iota(jnp.int32, _L)
    ones = jnp.ones((_L,), jnp.int32)
    zeros = jnp.zeros((_L,), jnp.int32)
    for i in range(_HB // _L):
        hist_v[pl.ds(i * _L, _L)] = zeros
    for i in range(_NG):
        cv = cnt_v[pl.ds(i * _L, _L)]
        sv = lax.rem(cv, _CAP)
        slot_v[pl.ds(i * _L, _L)] = sv
        cnt_v[pl.ds(i * _L, _L)] = cv + 1
        plsc.addupdate_scatter(hist_v, [sv], ones)
    pltpu.sync_copy(cnt_v, cnt_out_hbm.at[pl.ds(base, _BPW)])

    carry = jnp.int32(0)
    for i in range(4):
        h = hist_v[pl.ds(i * _L, _L)]
        c = plsc.cumsum(h)
        excl = c - h + carry
        offs_v[pl.ds(i * _L, _L)] = excl
        cur_v[pl.ds(i * _L, _L)] = excl
        carry = carry + c[_L - 1]

    lane0 = lanes == 0
    for gi in range(_NG):
        svg = slot_v[pl.ds(gi * _L, _L)]
        for l in range(_L):
            g = gi * _L + l
            s_g = svg[l]
            p = cur_v[pl.ds(s_g, _L)][0]
            plsc.store_scatter(list_v, [jnp.full((_L,), p, jnp.int32)],
                               jnp.full((_L,), g, jnp.int32), mask=lane0)
            plsc.store_scatter(cur_v, [jnp.full((_L,), s_g, jnp.int32)],
                               jnp.full((_L,), p + 1, jnp.int32),
                               mask=lane0)

    wrs = {}
    for s in range(_CAP):
        if s + _RA < _CAP:
            if s + _RA >= _NB:
                wrs[s + _RA - _NB].wait()
            rds[s + _RA] = rd(s + _RA)
        rds[s].wait()
        grp = (s // _L) * _L
        st = offs_v[pl.ds(grp, _L)][s % _L]
        n = hist_v[pl.ds(grp, _L)][s % _L]
        buf = bufs[s % _NB]

        def ov_body(t, c, st=st, buf=buf):
            g = list_v[pl.ds(st + t, _L)][0]
            for j in range(_MEM // _L):
                buf[g, pl.ds(j * _L, _L)] = rows_v[g, pl.ds(j * _L, _L)]
            return c

        lax.fori_loop(0, n, ov_body, jnp.int32(0))
        wrs[s] = wr(s)
    for s in range(_CAP - _NB, _CAP):
        wrs[s].wait()


_sc_write = pl.kernel(
    _sc_body,
    out_type=(jax.ShapeDtypeStruct((_CAP, _B, _MEM), jnp.float32),
              jax.ShapeDtypeStruct((_CAP, _B, _MEM), jnp.float32),
              jax.ShapeDtypeStruct((_B,), jnp.int32)),
    mesh=plsc.VectorSubcoreMesh(core_axis_name="c", subcore_axis_name="s",
                                num_cores=_NC, num_subcores=_NS),
    compiler_params=pltpu.CompilerParams(needs_layout_passes=False),
    scratch_types=[
        pltpu.VMEM((_BPW,), jnp.int32),
        pltpu.VMEM((_BPW,), jnp.int32),
        pltpu.VMEM((_BPW, _MEM), jnp.float32),
        pltpu.VMEM((_HB,), jnp.int32),
        pltpu.VMEM((_HB,), jnp.int32),
        pltpu.VMEM((_HB,), jnp.int32),
        pltpu.VMEM((_BPW + _L,), jnp.int32),
    ] + [pltpu.VMEM((_BPW, _MEM), jnp.float32)] * _NB
      + [pltpu.SemaphoreType.DMA] * (3 * _NB),
)


def kernel(inputs, cnt, mem):
    cnt = cnt.astype(jnp.int32)
    mem_t = mem.transpose(1, 0, 2)
    out_t, out2_t, counter = _sc_write(inputs, cnt, mem_t)
    memories = out_t.transpose(1, 0, 2)
    return (memories, counter, out2_t.transpose(1, 0, 2))


# speedup vs baseline: 24.8536x; 1.0104x over previous
import jax
import jax.numpy as jnp
from jax import lax
from jax.experimental import pallas as pl
from jax.experimental.pallas import tpu as pltpu
from jax.experimental.pallas import tpu_sc as plsc

_CAP = 50
_MEM = 128
_B = 4096

_NC = 2
_NS = 16
_NW = _NC * _NS
_BPW = _B // _NW
_L = 16
_NG = _BPW // _L
_HB = 80

_NB = 6
_RA = 3


def _sc_body(inputs_hbm, cnt_hbm, mem_hbm, out_hbm, out2_hbm, cnt_out_hbm,
             cnt_v, slot_v, rows_v, hist_v, offs_v, cur_v, list_v,
             buf0, buf1, buf2, buf3, buf4, buf5,
             rd_sem0, rd_sem1, rd_sem2, rd_sem3, rd_sem4, rd_sem5,
             wr_sem0, wr_sem1, wr_sem2, wr_sem3, wr_sem4, wr_sem5,
             w2_sem0, w2_sem1, w2_sem2, w2_sem3, w2_sem4, w2_sem5):
    wid = lax.axis_index("s") * _NC + lax.axis_index("c")
    base = wid * _BPW
    bufs = (buf0, buf1, buf2, buf3, buf4, buf5)
    rd_sems = (rd_sem0, rd_sem1, rd_sem2, rd_sem3, rd_sem4, rd_sem5)
    wr_sems = (wr_sem0, wr_sem1, wr_sem2, wr_sem3, wr_sem4, wr_sem5)
    w2_sems = (w2_sem0, w2_sem1, w2_sem2, w2_sem3, w2_sem4, w2_sem5)

    def rd(s):
        return pltpu.async_copy(
            mem_hbm.at[s, pl.ds(base, _BPW)], bufs[s % _NB],
            rd_sems[s % _NB])

    class _Wr2:
        def __init__(self, s):
            self.a = pltpu.async_copy(
                bufs[s % _NB], out_hbm.at[s, pl.ds(base, _BPW)],
                wr_sems[s % _NB])
            self.b = pltpu.async_copy(
                bufs[s % _NB], out2_hbm.at[s, pl.ds(base, _BPW)],
                w2_sems[s % _NB])

        def wait(self):
            self.a.wait()
            self.b.wait()

    wr = _Wr2

    rds = {s: rd(s) for s in range(_RA)}

    pltpu.sync_copy(cnt_hbm.at[pl.ds(base, _BPW)], cnt_v)
    pltpu.sync_copy(inputs_hbm.at[pl.ds(base, _BPW)], rows_v)
    lanes = lax.iota(jnp.int32, _L)
    ones = jnp.ones((_L,), jnp.int32)
    zeros = jnp.zeros((_L,), jnp.int32)
    for i in range(_HB // _L):
        hist_v[pl.ds(i * _L, _L)] = zeros
    for i in range(_NG):
        cv = cnt_v[pl.ds(i * _L, _L)]
        sv = lax.rem(cv, _CAP)
        slot_v[pl.ds(i * _L, _L)] = sv
        cnt_v[pl.ds(i * _L, _L)] = cv + 1
        plsc.addupdate_scatter(hist_v, [sv], ones)
    pltpu.sync_copy(cnt_v, cnt_out_hbm.at[pl.ds(base, _BPW)])

    carry = jnp.int32(0)
    for i in range(4):
        h = hist_v[pl.ds(i * _L, _L)]
        c = plsc.cumsum(h)
        excl = c - h + carry
        offs_v[pl.ds(i * _L, _L)] = excl
        cur_v[pl.ds(i * _L, _L)] = excl
        carry = carry + c[_L - 1]

    lane0 = lanes == 0
    for gi in range(_NG):
        svg = slot_v[pl.ds(gi * _L, _L)]
        for l in range(_L):
            g = gi * _L + l
            s_g = svg[l]
            p = cur_v[pl.ds(s_g, _L)][0]
            plsc.store_scatter(list_v, [jnp.full((_L,), p, jnp.int32)],
                               jnp.full((_L,), g, jnp.int32), mask=lane0)
            plsc.store_scatter(cur_v, [jnp.full((_L,), s_g, jnp.int32)],
                               jnp.full((_L,), p + 1, jnp.int32),
                               mask=lane0)

    wrs = {}
    for s in range(_CAP):
        if s + _RA < _CAP:
            if s + _RA >= _NB:
                wrs[s + _RA - _NB].wait()
            rds[s + _RA] = rd(s + _RA)
        rds[s].wait()
        grp = (s // _L) * _L
        st = offs_v[pl.ds(grp, _L)][s % _L]
        n = hist_v[pl.ds(grp, _L)][s % _L]
        buf = bufs[s % _NB]

        def ov_body(t, c, st=st, buf=buf):
            g = list_v[pl.ds(st + t, _L)][0]
            for j in range(_MEM // _L):
                buf[g, pl.ds(j * _L, _L)] = rows_v[g, pl.ds(j * _L, _L)]
            return c

        lax.fori_loop(0, n, ov_body, jnp.int32(0))
        wrs[s] = wr(s)
    for s in range(_CAP - _NB, _CAP):
        wrs[s].wait()


_sc_write = pl.kernel(
    _sc_body,
    out_type=(jax.ShapeDtypeStruct((_CAP, _B, _MEM), jnp.float32),
              jax.ShapeDtypeStruct((_CAP, _B, _MEM), jnp.float32),
              jax.ShapeDtypeStruct((_B,), jnp.int32)),
    mesh=plsc.VectorSubcoreMesh(core_axis_name="c", subcore_axis_name="s",
                                num_cores=_NC, num_subcores=_NS),
    compiler_params=pltpu.CompilerParams(needs_layout_passes=False),
    scratch_types=[
        pltpu.VMEM((_BPW,), jnp.int32),
        pltpu.VMEM((_BPW,), jnp.int32),
        pltpu.VMEM((_BPW, _MEM), jnp.float32),
        pltpu.VMEM((_HB,), jnp.int32),
        pltpu.VMEM((_HB,), jnp.int32),
        pltpu.VMEM((_HB,), jnp.int32),
        pltpu.VMEM((_BPW + _L,), jnp.int32),
    ] + [pltpu.VMEM((_BPW, _MEM), jnp.float32)] * _NB
      + [pltpu.SemaphoreType.DMA] * (3 * _NB),
)


def kernel(inputs, cnt, mem):
    cnt = cnt.astype(jnp.int32)
    mem_t = mem.transpose(1, 0, 2)
    out_t, out2_t, counter = _sc_write(inputs, cnt, mem_t)
    memories = out_t.transpose(1, 0, 2)
    return (memories, counter, out2_t.transpose(1, 0, 2))


# speedup vs baseline: 25.0970x; 1.0098x over previous
import jax
import jax.numpy as jnp
from jax import lax
from jax.experimental import pallas as pl
from jax.experimental.pallas import tpu as pltpu
from jax.experimental.pallas import tpu_sc as plsc

_CAP = 50
_MEM = 128
_B = 4096

_NC = 2
_NS = 16
_NW = _NC * _NS
_BPW = _B // _NW
_L = 16
_NG = _BPW // _L
_HB = 80

_NB = 6
_RA = 2


def _sc_body(inputs_hbm, cnt_hbm, mem_hbm, out_hbm, out2_hbm, cnt_out_hbm,
             cnt_v, slot_v, rows_v, hist_v, offs_v, cur_v, list_v,
             buf0, buf1, buf2, buf3, buf4, buf5,
             rd_sem0, rd_sem1, rd_sem2, rd_sem3, rd_sem4, rd_sem5,
             wr_sem0, wr_sem1, wr_sem2, wr_sem3, wr_sem4, wr_sem5,
             w2_sem0, w2_sem1, w2_sem2, w2_sem3, w2_sem4, w2_sem5):
    wid = lax.axis_index("s") * _NC + lax.axis_index("c")
    base = wid * _BPW
    bufs = (buf0, buf1, buf2, buf3, buf4, buf5)
    rd_sems = (rd_sem0, rd_sem1, rd_sem2, rd_sem3, rd_sem4, rd_sem5)
    wr_sems = (wr_sem0, wr_sem1, wr_sem2, wr_sem3, wr_sem4, wr_sem5)
    w2_sems = (w2_sem0, w2_sem1, w2_sem2, w2_sem3, w2_sem4, w2_sem5)

    def rd(s):
        return pltpu.async_copy(
            mem_hbm.at[s, pl.ds(base, _BPW)], bufs[s % _NB],
            rd_sems[s % _NB])

    class _Wr2:
        def __init__(self, s):
            self.a = pltpu.async_copy(
                bufs[s % _NB], out_hbm.at[s, pl.ds(base, _BPW)],
                wr_sems[s % _NB])
            self.b = pltpu.async_copy(
                bufs[s % _NB], out2_hbm.at[s, pl.ds(base, _BPW)],
                w2_sems[s % _NB])

        def wait(self):
            self.a.wait()
            self.b.wait()

    wr = _Wr2

    rds = {s: rd(s) for s in range(_RA)}

    pltpu.sync_copy(cnt_hbm.at[pl.ds(base, _BPW)], cnt_v)
    pltpu.sync_copy(inputs_hbm.at[pl.ds(base, _BPW)], rows_v)
    lanes = lax.iota(jnp.int32, _L)
    ones = jnp.ones((_L,), jnp.int32)
    zeros = jnp.zeros((_L,), jnp.int32)
    for i in range(_HB // _L):
        hist_v[pl.ds(i * _L, _L)] = zeros
    for i in range(_NG):
        cv = cnt_v[pl.ds(i * _L, _L)]
        sv = lax.rem(cv, _CAP)
        slot_v[pl.ds(i * _L, _L)] = sv
        cnt_v[pl.ds(i * _L, _L)] = cv + 1
        plsc.addupdate_scatter(hist_v, [sv], ones)
    pltpu.sync_copy(cnt_v, cnt_out_hbm.at[pl.ds(base, _BPW)])

    carry = jnp.int32(0)
    for i in range(4):
        h = hist_v[pl.ds(i * _L, _L)]
        c = plsc.cumsum(h)
        excl = c - h + carry
        offs_v[pl.ds(i * _L, _L)] = excl
        cur_v[pl.ds(i * _L, _L)] = excl
        carry = carry + c[_L - 1]

    lane0 = lanes == 0
    for gi in range(_NG):
        svg = slot_v[pl.ds(gi * _L, _L)]
        for l in range(_L):
            g = gi * _L + l
            s_g = svg[l]
            p = cur_v[pl.ds(s_g, _L)][0]
            plsc.store_scatter(list_v, [jnp.full((_L,), p, jnp.int32)],
                               jnp.full((_L,), g, jnp.int32), mask=lane0)
            plsc.store_scatter(cur_v, [jnp.full((_L,), s_g, jnp.int32)],
                               jnp.full((_L,), p + 1, jnp.int32),
                               mask=lane0)

    wrs = {}
    for s in range(_CAP):
        if s + _RA < _CAP:
            if s + _RA >= _NB:
                wrs[s + _RA - _NB].wait()
            rds[s + _RA] = rd(s + _RA)
        rds[s].wait()
        grp = (s // _L) * _L
        st = offs_v[pl.ds(grp, _L)][s % _L]
        n = hist_v[pl.ds(grp, _L)][s % _L]
        buf = bufs[s % _NB]

        def ov_body(t, c, st=st, buf=buf):
            g = list_v[pl.ds(st + t, _L)][0]
            for j in range(_MEM // _L):
                buf[g, pl.ds(j * _L, _L)] = rows_v[g, pl.ds(j * _L, _L)]
            return c

        lax.fori_loop(0, n, ov_body, jnp.int32(0))
        wrs[s] = wr(s)
    for s in range(_CAP - _NB, _CAP):
        wrs[s].wait()


_sc_write = pl.kernel(
    _sc_body,
    out_type=(jax.ShapeDtypeStruct((_CAP, _B, _MEM), jnp.float32),
              jax.ShapeDtypeStruct((_CAP, _B, _MEM), jnp.float32),
              jax.ShapeDtypeStruct((_B,), jnp.int32)),
    mesh=plsc.VectorSubcoreMesh(core_axis_name="c", subcore_axis_name="s",
                                num_cores=_NC, num_subcores=_NS),
    compiler_params=pltpu.CompilerParams(needs_layout_passes=False),
    scratch_types=[
        pltpu.VMEM((_BPW,), jnp.int32),
        pltpu.VMEM((_BPW,), jnp.int32),
        pltpu.VMEM((_BPW, _MEM), jnp.float32),
        pltpu.VMEM((_HB,), jnp.int32),
        pltpu.VMEM((_HB,), jnp.int32),
        pltpu.VMEM((_HB,), jnp.int32),
        pltpu.VMEM((_BPW + _L,), jnp.int32),
    ] + [pltpu.VMEM((_BPW, _MEM), jnp.float32)] * _NB
      + [pltpu.SemaphoreType.DMA] * (3 * _NB),
)


def kernel(inputs, cnt, mem):
    cnt = cnt.astype(jnp.int32)
    mem_t = mem.transpose(1, 0, 2)
    out_t, out2_t, counter = _sc_write(inputs, cnt, mem_t)
    memories = out_t.transpose(1, 0, 2)
    return (memories, counter, out2_t.transpose(1, 0, 2))
